# baseline (speedup 1.0000x reference)
import jax
import jax.numpy as jnp
from jax.experimental import pallas as pl
from jax.experimental.pallas import tpu as pltpu

_ALPHA = 0.1
_K = 10
_STILE = 512
_TTILE = 256


def _appnp_body(x_ref, adj_win_ref, o_ref, ab_ref, hf_ref, hb_ref):
    i = pl.program_id(0)
    n = x_ref.shape[0]
    num_blocks = n // _STILE
    sl = pl.ds(i * _STILE, _STILE)

    @pl.when(i == 0)
    def _init():
        hb_ref[...] = x_ref[...].astype(jnp.bfloat16)

    ab_ref[sl, :] = adj_win_ref[...].astype(jnp.bfloat16)
    ah0 = jnp.dot(ab_ref[sl, :], hb_ref[...],
                  preferred_element_type=jnp.float32)
    hf_ref[sl, :] = (1.0 - _ALPHA) * ah0 + _ALPHA * x_ref[sl, :]

    @pl.when(i == num_blocks - 1)
    def _tail():
        num_tiles = n // _TTILE

        def step(_, carry):
            hb_ref[...] = hf_ref[...].astype(jnp.bfloat16)
            for t in range(num_tiles):
                tsl = pl.ds(t * _TTILE, _TTILE)
                ah = jnp.dot(ab_ref[tsl, :], hb_ref[...],
                             preferred_element_type=jnp.float32)
                hf_ref[tsl, :] = (1.0 - _ALPHA) * ah + _ALPHA * x_ref[tsl, :]
            return carry

        jax.lax.fori_loop(0, _K - 2, step, 0)

        hb_ref[...] = hf_ref[...].astype(jnp.bfloat16)
        for t in range(num_tiles):
            tsl = pl.ds(t * _TTILE, _TTILE)
            ah = jnp.dot(ab_ref[tsl, :], hb_ref[...],
                         preferred_element_type=jnp.float32)
            o_ref[tsl, :] = (1.0 - _ALPHA) * ah + _ALPHA * x_ref[tsl, :]


def kernel(x, adj):
    n, f = x.shape
    num_blocks = n // _STILE
    return pl.pallas_call(
        _appnp_body,
        grid=(num_blocks,),
        in_specs=[
            pl.BlockSpec((n, f), lambda i: (0, 0)),
            pl.BlockSpec((_STILE, n), lambda i: (i, 0)),
        ],
        out_specs=pl.BlockSpec((n, f), lambda i: (0, 0)),
        out_shape=jax.ShapeDtypeStruct(x.shape, x.dtype),
        scratch_shapes=[
            pltpu.VMEM((n, n), jnp.bfloat16),
            pltpu.VMEM((n, f), jnp.float32),
            pltpu.VMEM((n, f), jnp.bfloat16),
        ],
        compiler_params=pltpu.CompilerParams(
            vmem_limit_bytes=64 * 1024 * 1024,
        ),
    )(x, adj)


# speedup vs baseline: 1.0637x; 1.0637x over previous
import jax
import jax.numpy as jnp
from jax.experimental import pallas as pl
from jax.experimental.pallas import tpu as pltpu

_ALPHA = 0.1
_K = 10
_STILE = 512
_TTILE = 512


def _appnp_body(x_ref, adj_win_ref, o_ref, ab_ref, xb_ref, ha_ref, hb_ref):
    i = pl.program_id(0)
    n = x_ref.shape[0]
    num_blocks = n // _STILE
    sl = pl.ds(i * _STILE, _STILE)

    @pl.when(i == 0)
    def _init():
        xb_ref[...] = x_ref[...].astype(jnp.bfloat16)

    ab_ref[sl, :] = adj_win_ref[...].astype(jnp.bfloat16)
    ah0 = jnp.dot(ab_ref[sl, :], xb_ref[...],
                  preferred_element_type=jnp.float32)
    ha_ref[sl, :] = ((1.0 - _ALPHA) * ah0
                     + _ALPHA * x_ref[sl, :]).astype(jnp.bfloat16)

    @pl.when(i == num_blocks - 1)
    def _tail():
        num_tiles = n // _TTILE

        def one_step(src_ref, dst_ref):
            for t in range(num_tiles):
                tsl = pl.ds(t * _TTILE, _TTILE)
                ah = jnp.dot(ab_ref[tsl, :], src_ref[...],
                             preferred_element_type=jnp.float32)
                res = (1.0 - _ALPHA) * ah + _ALPHA * x_ref[tsl, :]
                dst_ref[tsl, :] = res.astype(jnp.bfloat16)

        def pair(_, carry):
            one_step(ha_ref, hb_ref)
            one_step(hb_ref, ha_ref)
            return carry

        jax.lax.fori_loop(0, (_K - 2) // 2, pair, 0)

        for t in range(num_tiles):
            tsl = pl.ds(t * _TTILE, _TTILE)
            ah = jnp.dot(ab_ref[tsl, :], ha_ref[...],
                         preferred_element_type=jnp.float32)
            o_ref[tsl, :] = (1.0 - _ALPHA) * ah + _ALPHA * x_ref[tsl, :]


def kernel(x, adj):
    n, f = x.shape
    num_blocks = n // _STILE
    return pl.pallas_call(
        _appnp_body,
        grid=(num_blocks,),
        in_specs=[
            pl.BlockSpec((n, f), lambda i: (0, 0)),
            pl.BlockSpec((_STILE, n), lambda i: (i, 0)),
        ],
        out_specs=pl.BlockSpec((n, f), lambda i: (0, 0)),
        out_shape=jax.ShapeDtypeStruct(x.shape, x.dtype),
        scratch_shapes=[
            pltpu.VMEM((n, n), jnp.bfloat16),
            pltpu.VMEM((n, f), jnp.bfloat16),
            pltpu.VMEM((n, f), jnp.bfloat16),
            pltpu.VMEM((n, f), jnp.bfloat16),
        ],
        compiler_params=pltpu.CompilerParams(
            vmem_limit_bytes=64 * 1024 * 1024,
        ),
    )(x, adj)
